# baseline (speedup 1.0000x reference)
import functools

import jax
import jax.numpy as jnp
from jax import lax
from jax.experimental import pallas as pl
from jax.experimental.pallas import tpu as pltpu
from jax.experimental.pallas import tpu_sc as plsc

_B = 64
_C = 3
_H = 128
_NSLOT = 7


def kernel(x, t, lr_up, t_embeddings):
    info = plsc.get_sparse_core_info()
    nc = info.num_cores
    nw = nc * info.num_subcores
    b_per_w = _B // nw
    rows_per_w = _C * b_per_w
    idx_pad = jnp.pad(t.astype(jnp.int32).reshape(nw, b_per_w, 1),
                      ((0, 0), (0, 0), (0, 7))).reshape(nw, 8 * b_per_w)

    mesh = plsc.VectorSubcoreMesh(core_axis_name="c", subcore_axis_name="s")

    @functools.partial(
        pl.kernel,
        out_type=jax.ShapeDtypeStruct((_B, 1 + _C, _H, _H), jnp.float32),
        mesh=mesh,
        compiler_params=pltpu.CompilerParams(skip_device_barrier=True),
        scratch_types=[
            pltpu.VMEM((16,), jnp.int32),
            pltpu.VMEM((_NSLOT, 1, _H, _H), jnp.float32),
            pltpu.SemaphoreType.DMA((b_per_w,)),
            pltpu.SemaphoreType.DMA((_NSLOT,)),
            pltpu.SemaphoreType.DMA((_NSLOT + 1,)),
        ],
    )
    def sc_kernel(table_hbm, idx_hbm, lr_hbm, out_hbm, idx_v, pool, gsems,
                  isems, osems):
        wid = lax.axis_index("s") * nc + lax.axis_index("c")
        base = wid * b_per_w
        pool_flat = pool.reshape(_NSLOT, _H * _H)

        def lr_src(r):
            return lr_hbm.at[pl.ds(base + r // _C, 1), pl.ds(r % _C, 1)]

        def lr_dst(r):
            return out_hbm.at[pl.ds(base + r // _C, 1), pl.ds(1 + r % _C, 1)]

        in_cp = {}
        for r in range(_NSLOT - b_per_w):
            in_cp[r] = pltpu.async_copy(
                lr_src(r), pool.at[pl.ds(b_per_w + r, 1)],
                isems.at[b_per_w + r])
        pltpu.sync_copy(idx_hbm.at[wid], idx_v)
        gcp = [
            pltpu.async_copy(
                table_hbm.at[idx_v.at[pl.ds(8 * i, 1)]],
                pool_flat.at[pl.ds(i, 1)], gsems.at[i])
            for i in range(b_per_w)
        ]
        gout = []
        for i in range(b_per_w):
            gcp[i].wait()
            gout.append(pltpu.async_copy(
                pool.at[pl.ds(i, 1)],
                out_hbm.at[pl.ds(base + i, 1), pl.ds(0, 1)], osems.at[i]))
        gout[0].wait()
        in_cp[rows_per_w - 1] = pltpu.async_copy(
            lr_src(rows_per_w - 1), pool.at[pl.ds(0, 1)], isems.at[0])
        out_cp = {}
        for r in range(rows_per_w):
            slot = (b_per_w + r) % _NSLOT
            in_cp[r].wait()
            out_cp[r] = pltpu.async_copy(
                pool.at[pl.ds(slot, 1)], lr_dst(r), osems.at[b_per_w + r])
        for r in range(rows_per_w):
            out_cp[r].wait()
        gout[1].wait()

    return sc_kernel(t_embeddings, idx_pad, lr_up)


# speedup vs baseline: 1.0024x; 1.0024x over previous
import functools

import jax
import jax.numpy as jnp
from jax import lax
from jax.experimental import pallas as pl
from jax.experimental.pallas import tpu as pltpu
from jax.experimental.pallas import tpu_sc as plsc

_B = 64
_C = 3
_H = 128
_NSLOT = 7


def kernel(x, t, lr_up, t_embeddings):
    info = plsc.get_sparse_core_info()
    nc = info.num_cores
    nw = nc * info.num_subcores
    b_per_w = _B // nw
    rows_per_w = _C * b_per_w
    idx_pad = jnp.pad(t.astype(jnp.int32).reshape(nw, b_per_w, 1),
                      ((0, 0), (0, 0), (0, 7))).reshape(nw, 8 * b_per_w)

    mesh = plsc.VectorSubcoreMesh(core_axis_name="c", subcore_axis_name="s")

    @functools.partial(
        pl.kernel,
        out_type=jax.ShapeDtypeStruct((_B, 1 + _C, _H, _H), jnp.float32),
        mesh=mesh,
        compiler_params=pltpu.CompilerParams(skip_device_barrier=True),
        scratch_types=[
            pltpu.VMEM((16,), jnp.int32),
            pltpu.VMEM((_NSLOT, 1, _H, _H), jnp.float32),
            pltpu.SemaphoreType.DMA((b_per_w,)),
            pltpu.SemaphoreType.DMA((_NSLOT,)),
            pltpu.SemaphoreType.DMA((_NSLOT + 1,)),
        ],
    )
    def sc_kernel(table_hbm, idx_hbm, lr_hbm, out_hbm, idx_v, pool, gsems,
                  isems, osems):
        wid = lax.axis_index("s") * nc + lax.axis_index("c")
        base = wid * b_per_w
        pool_flat = pool.reshape(_NSLOT, _H * _H)

        def lr_src(r):
            return lr_hbm.at[pl.ds(base + r // _C, 1), pl.ds(r % _C, 1)]

        def lr_dst(r):
            return out_hbm.at[pl.ds(base + r // _C, 1), pl.ds(1 + r % _C, 1)]

        icp = pltpu.async_copy(idx_hbm.at[wid], idx_v, isems.at[0])
        in_cp = {}
        for r in range(_NSLOT - b_per_w):
            in_cp[r] = pltpu.async_copy(
                lr_src(r), pool.at[pl.ds(b_per_w + r, 1)],
                isems.at[b_per_w + r])
        icp.wait()
        gcp = [
            pltpu.async_copy(
                table_hbm.at[idx_v.at[pl.ds(8 * i, 1)]],
                pool_flat.at[pl.ds(i, 1)], gsems.at[i])
            for i in range(b_per_w)
        ]
        out_cp = {}
        for r in range(rows_per_w - 1):
            in_cp[r].wait()
            out_cp[r] = pltpu.async_copy(
                pool.at[pl.ds(b_per_w + r, 1)], lr_dst(r),
                osems.at[b_per_w + r])
        gout = []
        for i in range(b_per_w):
            gcp[i].wait()
            gout.append(pltpu.async_copy(
                pool.at[pl.ds(i, 1)],
                out_hbm.at[pl.ds(base + i, 1), pl.ds(0, 1)], osems.at[i]))
        out_cp[0].wait()
        last = rows_per_w - 1
        in_cp[last] = pltpu.async_copy(
            lr_src(last), pool.at[pl.ds(b_per_w, 1)], isems.at[0])
        in_cp[last].wait()
        out_cp[last] = pltpu.async_copy(
            pool.at[pl.ds(b_per_w, 1)], lr_dst(last), osems.at[b_per_w + last])
        for r in range(1, rows_per_w):
            out_cp[r].wait()
        gout[0].wait()
        gout[1].wait()

    return sc_kernel(t_embeddings, idx_pad, lr_up)


# speedup vs baseline: 1.0055x; 1.0031x over previous
import functools

import jax
import jax.numpy as jnp
from jax import lax
from jax.experimental import pallas as pl
from jax.experimental.pallas import tpu as pltpu
from jax.experimental.pallas import tpu_sc as plsc

_B = 64
_C = 3
_H = 128
_NSLOT = 7


def kernel(x, t, lr_up, t_embeddings):
    info = plsc.get_sparse_core_info()
    nc = info.num_cores
    nw = nc * info.num_subcores
    b_per_w = _B // nw
    rows_per_w = _C * b_per_w

    mesh = plsc.VectorSubcoreMesh(core_axis_name="c", subcore_axis_name="s")

    @functools.partial(
        pl.kernel,
        out_type=jax.ShapeDtypeStruct((_B, 1 + _C, _H, _H), jnp.float32),
        mesh=mesh,
        compiler_params=pltpu.CompilerParams(needs_layout_passes=False),
        scratch_types=[
            pltpu.VMEM((_B,), jnp.int32),
            pltpu.VMEM((16,), jnp.int32),
            pltpu.VMEM((_NSLOT, 1, _H, _H), jnp.float32),
            pltpu.SemaphoreType.DMA((b_per_w,)),
            pltpu.SemaphoreType.DMA((_NSLOT,)),
            pltpu.SemaphoreType.DMA((_NSLOT + 1,)),
        ],
    )
    def sc_kernel(table_hbm, idx_hbm, lr_hbm, out_hbm, idx_all, idx_v, pool,
                  gsems, isems, osems):
        wid = lax.axis_index("s") * nc + lax.axis_index("c")
        base = wid * b_per_w
        pool_flat = pool.reshape(_NSLOT, _H * _H)

        def lr_src(r):
            return lr_hbm.at[pl.ds(base + r // _C, 1), pl.ds(r % _C, 1)]

        def lr_dst(r):
            return out_hbm.at[pl.ds(base + r // _C, 1), pl.ds(1 + r % _C, 1)]

        icp = pltpu.async_copy(idx_hbm, idx_all, isems.at[0])
        in_cp = {}
        for r in range(_NSLOT - b_per_w):
            in_cp[r] = pltpu.async_copy(
                lr_src(r), pool.at[pl.ds(b_per_w + r, 1)],
                isems.at[b_per_w + r])
        icp.wait()
        lane = lax.iota(jnp.int32, 16)
        group = wid // (16 // b_per_w)
        l0 = b_per_w * (wid % (16 // b_per_w))
        vals = idx_all[pl.ds(16 * group, 16)]
        sel0 = lane == l0
        sel1 = lane == l0 + 1
        pos = jnp.where(sel0, 0, jnp.where(sel1, 8, 15))
        plsc.store_scatter(idx_v, [pos], vals, mask=sel0 | sel1)
        gcp = [
            pltpu.async_copy(
                table_hbm.at[idx_v.at[pl.ds(8 * i, 1)]],
                pool_flat.at[pl.ds(i, 1)], gsems.at[i])
            for i in range(b_per_w)
        ]
        out_cp = {}
        for r in range(rows_per_w - 1):
            in_cp[r].wait()
            out_cp[r] = pltpu.async_copy(
                pool.at[pl.ds(b_per_w + r, 1)], lr_dst(r),
                osems.at[b_per_w + r])
        gout = []
        for i in range(b_per_w):
            gcp[i].wait()
            gout.append(pltpu.async_copy(
                pool.at[pl.ds(i, 1)],
                out_hbm.at[pl.ds(base + i, 1), pl.ds(0, 1)], osems.at[i]))
        out_cp[0].wait()
        last = rows_per_w - 1
        in_cp[last] = pltpu.async_copy(
            lr_src(last), pool.at[pl.ds(b_per_w, 1)], isems.at[0])
        in_cp[last].wait()
        out_cp[last] = pltpu.async_copy(
            pool.at[pl.ds(b_per_w, 1)], lr_dst(last), osems.at[b_per_w + last])
        for r in range(1, rows_per_w):
            out_cp[r].wait()
        gout[0].wait()
        gout[1].wait()

    return sc_kernel(t_embeddings, t.astype(jnp.int32), lr_up)


# speedup vs baseline: 1.0230x; 1.0175x over previous
import functools

import jax
import jax.numpy as jnp
from jax import lax
from jax.experimental import pallas as pl
from jax.experimental.pallas import tpu as pltpu
from jax.experimental.pallas import tpu_sc as plsc

_B = 64
_C = 3
_H = 128


def kernel(x, t, lr_up, t_embeddings):
    info = plsc.get_sparse_core_info()
    nc = info.num_cores
    nw = nc * info.num_subcores
    b_per_w = _B // nw
    idx_pad = jnp.pad(t.astype(jnp.int32).reshape(nw, b_per_w, 1),
                      ((0, 0), (0, 0), (0, 7))).reshape(nw, 8 * b_per_w)

    mesh = plsc.VectorSubcoreMesh(core_axis_name="c", subcore_axis_name="s")

    @functools.partial(
        pl.kernel,
        out_type=jax.ShapeDtypeStruct((_B, 1 + _C, _H, _H), jnp.float32),
        mesh=mesh,
        scratch_types=[
            pltpu.VMEM((16,), jnp.int32),
            pltpu.VMEM((2 * _C + 1, _H, _H), jnp.float32),
            pltpu.SemaphoreType.DMA((b_per_w,)),
            pltpu.SemaphoreType.DMA((b_per_w + 1,)),
            pltpu.SemaphoreType.DMA((2 * b_per_w,)),
        ],
    )
    def sc_kernel(table_hbm, idx_hbm, lr_hbm, out_hbm, idx_v, pool, gsems,
                  isems, osems):
        wid = lax.axis_index("s") * nc + lax.axis_index("c")
        b0 = wid * b_per_w
        pool_flat = pool.reshape(2 * _C + 1, _H * _H)
        lr3 = lr_hbm.reshape(_B * _C, _H, _H)
        out3 = out_hbm.reshape(_B * (1 + _C), _H, _H)

        icp = pltpu.async_copy(idx_hbm.at[wid], idx_v, isems.at[b_per_w])
        in_cp = [
            pltpu.async_copy(
                lr3.at[pl.ds(_C * (b0 + i), _C)],
                pool.at[pl.ds(_C * i, _C)], isems.at[i])
            for i in range(b_per_w)
        ]
        icp.wait()
        gather_slot = pool_flat.at[pl.ds(2 * _C, 1)]
        gout = []
        out_cp = []
        for i in range(b_per_w):
            pltpu.async_copy(
                table_hbm.at[idx_v.at[pl.ds(8 * i, 1)]], gather_slot,
                gsems.at[i]).wait()
            gout.append(pltpu.async_copy(
                pool.at[pl.ds(2 * _C, 1)],
                out3.at[pl.ds((1 + _C) * (b0 + i), 1)], osems.at[i]))
            if i + 1 < b_per_w:
                gout[i].wait()
            in_cp[i].wait()
            out_cp.append(pltpu.async_copy(
                pool.at[pl.ds(_C * i, _C)],
                out3.at[pl.ds((1 + _C) * (b0 + i) + 1, _C)],
                osems.at[b_per_w + i]))
        for cp in out_cp:
            cp.wait()
        gout[-1].wait()

    return sc_kernel(t_embeddings, idx_pad, lr_up)
